# Initial kernel scaffold; baseline (speedup 1.0000x reference)
import jax
import jax.numpy as jnp
from jax.experimental import pallas as pl


def kernel(x, edge_index, batch, coord, W, b, gamma, beta):
    raise NotImplementedError("write your pallas kernel here")


# speedup vs baseline: 3.4035x; 3.4035x over previous
import functools

import jax
import jax.numpy as jnp
from jax import lax
from jax.experimental import pallas as pl
from jax.experimental.pallas import tpu as pltpu
from jax.experimental.pallas import tpu_sc as plsc

N = 10000
D = 128
B = 64
C = 10

NC = 2
NS = 16
NW = NC * NS
CHUNK = 312
REM = N - NW * CHUNK
LANES = 16
DV = D // LANES


def _seg_body(x_hbm, b_hbm, sums_hbm, cnts_hbm, xv, bv, acc, cnt):
    wid = lax.axis_index("s") * NC + lax.axis_index("c")
    base = wid * CHUNK

    pltpu.sync_copy(x_hbm.at[pl.ds(base, CHUNK)], xv.at[pl.ds(0, CHUNK)])
    pltpu.sync_copy(b_hbm.at[pl.ds(base, CHUNK)], bv.at[pl.ds(0, CHUNK)])

    @pl.when(wid == NW - 1)
    def _():
        pltpu.sync_copy(x_hbm.at[pl.ds(NW * CHUNK, REM)], xv.at[pl.ds(CHUNK, REM)])
        pltpu.sync_copy(b_hbm.at[pl.ds(NW * CHUNK, REM)], bv.at[pl.ds(CHUNK, REM)])

    zeros = jnp.zeros((LANES,), jnp.float32)

    def zero_row(r, _):
        for j in range(DV):
            acc[r, pl.ds(j * LANES, LANES)] = zeros
        cnt[r, :] = zeros
        return 0

    lax.fori_loop(0, B, zero_row, 0)

    ones = jnp.ones((LANES,), jnp.float32)

    def do_row(s, i):
        for j in range(DV):
            plsc.addupdate(acc.at[s, pl.ds(j * LANES, LANES)],
                           xv[i, pl.ds(j * LANES, LANES)])
        plsc.addupdate(cnt.at[s], ones)

    nfull = CHUNK // LANES
    tail = CHUNK - nfull * LANES

    def group_body(g, _):
        r0 = g * LANES
        segv = bv[pl.ds(r0, LANES)]
        for k in range(LANES):
            do_row(segv[k], r0 + k)
        return 0

    lax.fori_loop(0, nfull, group_body, 0)

    segv = bv[pl.ds(nfull * LANES, LANES)]
    for k in range(tail):
        do_row(segv[k], nfull * LANES + k)

    @pl.when(wid == NW - 1)
    def _():
        segv2 = bv[pl.ds(CHUNK, LANES)]
        for k in range(REM):
            do_row(segv2[k], CHUNK + k)

    pltpu.sync_copy(acc, sums_hbm.at[wid])
    pltpu.sync_copy(cnt, cnts_hbm.at[wid])


@jax.jit
def _seg_pool(x, batch32):
    mesh = plsc.VectorSubcoreMesh(core_axis_name="c", subcore_axis_name="s")
    fn = functools.partial(
        pl.kernel,
        mesh=mesh,
        out_type=[
            jax.ShapeDtypeStruct((NW, B, D), jnp.float32),
            jax.ShapeDtypeStruct((NW, B, LANES), jnp.float32),
        ],
        scratch_types=[
            pltpu.VMEM((CHUNK + REM, D), jnp.float32),
            pltpu.VMEM((CHUNK + REM,), jnp.int32),
            pltpu.VMEM((B, D), jnp.float32),
            pltpu.VMEM((B, LANES), jnp.float32),
        ],
    )(_seg_body)
    return fn(x, batch32)


def _tail_body(sums_ref, cnts_ref, w_ref, b_ref, g_ref, beta_ref, o_ref):
    sums = jnp.sum(sums_ref[...], axis=0)
    counts = jnp.sum(cnts_ref[...], axis=0)[:, 0:1]
    mean = sums / jnp.clip(counts, 1.0, None)
    logits = jnp.dot(mean, w_ref[...].T,
                     preferred_element_type=jnp.float32) + b_ref[...]
    mu = jnp.mean(logits, axis=0, keepdims=True)
    var = jnp.mean((logits - mu) ** 2, axis=0, keepdims=True)
    o_ref[...] = (logits - mu) * lax.rsqrt(var + 1e-5) * g_ref[...] + beta_ref[...]


@jax.jit
def _tail(sums_p, cnts_p, W, b, gamma, beta):
    return pl.pallas_call(
        _tail_body,
        out_shape=jax.ShapeDtypeStruct((B, C), jnp.float32),
    )(sums_p, cnts_p, W, b.reshape(1, C), gamma.reshape(1, C), beta.reshape(1, C))


def kernel(x, edge_index, batch, coord, W, b, gamma, beta):
    del edge_index, coord
    batch32 = batch.astype(jnp.int32)
    sums_p, cnts_p = _seg_pool(x, batch32)
    return _tail(sums_p, cnts_p, W, b, gamma, beta)


# speedup vs baseline: 4.2479x; 1.2481x over previous
import functools

import jax
import jax.numpy as jnp
from jax import lax
from jax.experimental import pallas as pl
from jax.experimental.pallas import tpu as pltpu
from jax.experimental.pallas import tpu_sc as plsc

N = 10000
D = 128
B = 64
C = 10

NC = 2
NS = 16
NW = NC * NS
CHUNK = 312
REM = N - NW * CHUNK
LANES = 16
DV = D // LANES


def _seg_body(x_hbm, b_hbm, sums_hbm, cnts_hbm, xv, bv, acc, cnt, semx, semb):
    wid = lax.axis_index("s") * NC + lax.axis_index("c")
    base = wid * CHUNK

    cpx = pltpu.make_async_copy(x_hbm.at[pl.ds(base, CHUNK)],
                                xv.at[pl.ds(0, CHUNK)], semx)
    cpb = pltpu.make_async_copy(b_hbm.at[pl.ds(base, CHUNK)],
                                bv.at[pl.ds(0, CHUNK)], semb)
    cpx.start()
    cpb.start()

    zeros = jnp.zeros((LANES,), jnp.float32)

    def zero_row(r, _):
        for j in range(DV):
            acc[r, pl.ds(j * LANES, LANES)] = zeros
        cnt[r, :] = zeros
        return 0

    lax.fori_loop(0, B, zero_row, 0)

    cpx.wait()
    cpb.wait()

    @pl.when(wid == NW - 1)
    def _():
        pltpu.sync_copy(x_hbm.at[pl.ds(NW * CHUNK, REM)], xv.at[pl.ds(CHUNK, REM)])
        pltpu.sync_copy(b_hbm.at[pl.ds(NW * CHUNK, REM)], bv.at[pl.ds(CHUNK, REM)])

    def do_rows(r0, n, carry):
        cur, accs, cntf = carry
        segv = bv[pl.ds(r0, LANES)]
        for k in range(n):
            s = segv[k]
            fresh = s != cur
            keep = jnp.where(fresh, jnp.float32(0.0), jnp.float32(1.0))
            keepv = jnp.full((LANES,), keep)
            row = [xv[r0 + k, pl.ds(j * LANES, LANES)] for j in range(DV)]
            accs = tuple(row[j] + keepv * accs[j] for j in range(DV))
            cntf = 1.0 + keep * cntf
            for j in range(DV):
                acc[s, pl.ds(j * LANES, LANES)] = accs[j]
            cnt[s, :] = jnp.full((LANES,), cntf)
            cur = s
        return cur, accs, cntf

    carry0 = (jnp.int32(-1),
              tuple(jnp.zeros((LANES,), jnp.float32) for _ in range(DV)),
              jnp.float32(0.0))
    ngroups = jnp.where(wid == NW - 1, (CHUNK + REM) // LANES, CHUNK // LANES)

    def gbody(g, carry):
        return do_rows(g * LANES, LANES, carry)

    carry = lax.fori_loop(0, ngroups, gbody, carry0)
    do_rows(ngroups * LANES, CHUNK - (CHUNK // LANES) * LANES, carry)

    pltpu.sync_copy(acc, sums_hbm.at[wid])
    pltpu.sync_copy(cnt, cnts_hbm.at[wid])


@jax.jit
def _seg_pool(x, batch32):
    mesh = plsc.VectorSubcoreMesh(core_axis_name="c", subcore_axis_name="s")
    fn = functools.partial(
        pl.kernel,
        mesh=mesh,
        out_type=[
            jax.ShapeDtypeStruct((NW, B, D), jnp.float32),
            jax.ShapeDtypeStruct((NW, B, LANES), jnp.float32),
        ],
        scratch_types=[
            pltpu.VMEM((CHUNK + REM, D), jnp.float32),
            pltpu.VMEM((CHUNK + REM + LANES,), jnp.int32),
            pltpu.VMEM((B, D), jnp.float32),
            pltpu.VMEM((B, LANES), jnp.float32),
            pltpu.SemaphoreType.DMA,
            pltpu.SemaphoreType.DMA,
        ],
    )(_seg_body)
    return fn(x, batch32)


def _tail_body(sums_ref, cnts_ref, w_ref, b_ref, g_ref, beta_ref, o_ref):
    sums = jnp.sum(sums_ref[...], axis=0)
    counts = jnp.sum(cnts_ref[...], axis=0)[:, 0:1]
    mean = sums / jnp.clip(counts, 1.0, None)
    logits = jnp.dot(mean, w_ref[...].T,
                     preferred_element_type=jnp.float32) + b_ref[...]
    mu = jnp.mean(logits, axis=0, keepdims=True)
    var = jnp.mean((logits - mu) ** 2, axis=0, keepdims=True)
    o_ref[...] = (logits - mu) * lax.rsqrt(var + 1e-5) * g_ref[...] + beta_ref[...]


@jax.jit
def _tail(sums_p, cnts_p, W, b, gamma, beta):
    return pl.pallas_call(
        _tail_body,
        out_shape=jax.ShapeDtypeStruct((B, C), jnp.float32),
    )(sums_p, cnts_p, W, b.reshape(1, C), gamma.reshape(1, C), beta.reshape(1, C))


def kernel(x, edge_index, batch, coord, W, b, gamma, beta):
    del edge_index, coord
    batch32 = batch.astype(jnp.int32)
    sums_p, cnts_p = _seg_pool(x, batch32)
    return _tail(sums_p, cnts_p, W, b, gamma, beta)
